# baseline (speedup 1.0000x reference)
import functools

import jax
import jax.numpy as jnp
from jax.experimental import pallas as pl
from jax.experimental.pallas import tpu as pltpu

_DN = (((1,), (0,)), ((), ()))


def _pass1_body(x_ref, adj_ref, W1_ref, b1_ref, W2_ref, We_ref, be_ref,
                y_ref, s2_ref, adjbf_ref, s1_scr):
    i = pl.program_id(0)

    @pl.when(i == 0)
    def _():
        s1_scr[...] = jnp.dot(x_ref[...], W1_ref[...],
                              preferred_element_type=jnp.float32
                              ).astype(jnp.bfloat16)

    a = adj_ref[...]
    acc = jax.lax.dot_general(a, s1_scr[...], _DN,
                              preferred_element_type=jnp.float32)
    h = jnp.maximum(acc + b1_ref[...], 0.0)
    y_ref[...] = jnp.dot(h, We_ref[...],
                         preferred_element_type=jnp.float32) + be_ref[...]
    s2_ref[...] = jnp.dot(h, W2_ref[...], preferred_element_type=jnp.float32
                          ).astype(jnp.bfloat16)
    adjbf_ref[...] = a.astype(jnp.bfloat16)


def _pass2_body(adjbf_ref, s2_ref, b2_ref, logits_ref):
    z = jnp.dot(adjbf_ref[...], s2_ref[...],
                preferred_element_type=jnp.float32) + b2_ref[...]
    m = jnp.max(z, axis=1, keepdims=True)
    zs = z - m
    logits_ref[...] = zs - jnp.log(jnp.sum(jnp.exp(zs), axis=1,
                                           keepdims=True))


def kernel(x, adj, W1, b1, W2, b2, We, be):
    N, F = x.shape
    H = W1.shape[1]
    C = W2.shape[1]
    S = We.shape[1]
    BI = 400
    NI = N // BI

    y, s2, adjbf = pl.pallas_call(
        _pass1_body,
        grid=(NI,),
        in_specs=[
            pl.BlockSpec((N, F), lambda i: (0, 0)),
            pl.BlockSpec((BI, N), lambda i: (i, 0)),
            pl.BlockSpec((F, H), lambda i: (0, 0)),
            pl.BlockSpec((1, H), lambda i: (0, 0)),
            pl.BlockSpec((H, C), lambda i: (0, 0)),
            pl.BlockSpec((H, S), lambda i: (0, 0)),
            pl.BlockSpec((1, S), lambda i: (0, 0)),
        ],
        out_specs=[
            pl.BlockSpec((BI, S), lambda i: (i, 0)),
            pl.BlockSpec((BI, C), lambda i: (i, 0)),
            pl.BlockSpec((BI, N), lambda i: (i, 0)),
        ],
        out_shape=[
            jax.ShapeDtypeStruct((N, S), jnp.float32),
            jax.ShapeDtypeStruct((N, C), jnp.bfloat16),
            jax.ShapeDtypeStruct((N, N), jnp.bfloat16),
        ],
        scratch_shapes=[pltpu.VMEM((N, H), jnp.bfloat16)],
        compiler_params=pltpu.CompilerParams(
            dimension_semantics=("arbitrary",)),
    )(x.astype(jnp.bfloat16), adj, W1.astype(jnp.bfloat16),
      b1.reshape(1, H), W2, We, be.reshape(1, S))

    BB = 1000 if N % 1000 == 0 else BI
    NB = N // BB
    logits = pl.pallas_call(
        _pass2_body,
        grid=(NB,),
        in_specs=[
            pl.BlockSpec((BB, N), lambda i: (i, 0)),
            pl.BlockSpec((N, C), lambda i: (0, 0)),
            pl.BlockSpec((1, C), lambda i: (0, 0)),
        ],
        out_specs=pl.BlockSpec((BB, C), lambda i: (i, 0)),
        out_shape=jax.ShapeDtypeStruct((N, C), jnp.float32),
        compiler_params=pltpu.CompilerParams(
            dimension_semantics=("arbitrary",)),
    )(adjbf, s2, b2.reshape(1, C))

    return logits, y


# speedup vs baseline: 1.0858x; 1.0858x over previous
import functools

import jax
import jax.numpy as jnp
from jax.experimental import pallas as pl
from jax.experimental.pallas import tpu as pltpu

_DN = (((1,), (0,)), ((), ()))


def _gcn_body(x_ref, adj_ref, W1_ref, b1_ref, W2_ref, b2_ref, We_ref, be_ref,
              logits_ref, y_ref, s1_scr, s2_scr, *, BI):
    phase = pl.program_id(0)
    i = pl.program_id(1)

    @pl.when((phase == 0) & (i == 0))
    def _():
        s1_scr[...] = jnp.dot(x_ref[...], W1_ref[...],
                              preferred_element_type=jnp.float32
                              ).astype(jnp.bfloat16)

    @pl.when(phase == 0)
    def _():
        acc = jax.lax.dot_general(adj_ref[...], s1_scr[...], _DN,
                                  preferred_element_type=jnp.float32)
        h = jnp.maximum(acc + b1_ref[...], 0.0)
        y_ref[...] = jnp.dot(h, We_ref[...],
                             preferred_element_type=jnp.float32) + be_ref[...]
        s2_scr[pl.ds(i * BI, BI), :] = jnp.dot(
            h, W2_ref[...], preferred_element_type=jnp.float32
        ).astype(jnp.bfloat16)

    @pl.when(phase == 1)
    def _():
        z = jax.lax.dot_general(adj_ref[...], s2_scr[...], _DN,
                                preferred_element_type=jnp.float32
                                ) + b2_ref[...]
        m = jnp.max(z, axis=1, keepdims=True)
        zs = z - m
        logits_ref[...] = zs - jnp.log(jnp.sum(jnp.exp(zs), axis=1,
                                               keepdims=True))


def kernel(x, adj, W1, b1, W2, b2, We, be):
    N, F = x.shape
    H = W1.shape[1]
    C = W2.shape[1]
    S = We.shape[1]
    BI = 400
    NI = N // BI

    body = functools.partial(_gcn_body, BI=BI)

    out = pl.pallas_call(
        body,
        grid=(2, NI),
        in_specs=[
            pl.BlockSpec((N, F), lambda p, i: (0, 0)),
            pl.BlockSpec((BI, N), lambda p, i: (jnp.where(p == 0, i, NI - 1 - i), 0)),
            pl.BlockSpec((F, H), lambda p, i: (0, 0)),
            pl.BlockSpec((1, H), lambda p, i: (0, 0)),
            pl.BlockSpec((H, C), lambda p, i: (0, 0)),
            pl.BlockSpec((1, C), lambda p, i: (0, 0)),
            pl.BlockSpec((H, S), lambda p, i: (0, 0)),
            pl.BlockSpec((1, S), lambda p, i: (0, 0)),
        ],
        out_specs=[
            pl.BlockSpec((BI, C),
                         lambda p, i: (jnp.where(p == 1, NI - 1 - i, NI - 1), 0)),
            pl.BlockSpec((BI, S), lambda p, i: (jnp.where(p == 0, i, NI - 1), 0)),
        ],
        out_shape=[
            jax.ShapeDtypeStruct((N, C), jnp.float32),
            jax.ShapeDtypeStruct((N, S), jnp.float32),
        ],
        scratch_shapes=[
            pltpu.VMEM((N, H), jnp.bfloat16),
            pltpu.VMEM((N, C), jnp.bfloat16),
        ],
        compiler_params=pltpu.CompilerParams(
            dimension_semantics=("arbitrary", "arbitrary")),
    )(x, adj, W1, b1.reshape(1, H), W2, b2.reshape(1, C), We, be.reshape(1, S))
    return out[0], out[1]
